# baseline (speedup 1.0000x reference)
import jax
import jax.numpy as jnp
from jax import lax
from jax.experimental import pallas as pl
from jax.experimental.pallas import tpu as pltpu
from jax.experimental.pallas import tpu_sc as plsc

N = 10000
E = 320000
D = 128

NC = 2
NS = 16
NW = NC * NS
LANES = 16

G = 128
GROUPS = 80
IDX_CH = 8
EPW = G * GROUPS
E_PAD = EPW * NW
TOT_G = E_PAD // G
G2 = 64
TOT_G2 = E_PAD // G2
AIDX = 16
NBUF = 5
PRE = NBUF - 1
C0 = 224
C1 = 96

ACC_ROWS = 10112
RPT = ACC_ROWS // NS
DEG_ROWS = 16384
DPT = DEG_ROWS // NS
DINV_PAD = N + 16

RB = 1000


_MESH = plsc.VectorSubcoreMesh(core_axis_name="c", subcore_axis_name="s")
_SC_PARAMS = pltpu.CompilerParams(needs_layout_passes=False)


def _deg_body(dst_hbm, w_hbm, zrow_hbm, out_hbm, idx_v, w_v, acc):
    c = lax.axis_index("c")
    s = lax.axis_index("s")
    wid = c * NS + s
    tb = s * DPT
    pltpu.sync_copy(zrow_hbm.at[pl.ds(tb, DPT)], acc.at[pl.ds(tb, DPT)])
    pltpu.sync_copy(dst_hbm.at[pl.ds(wid * GROUPS, GROUPS), :], idx_v)
    pltpu.sync_copy(w_hbm.at[pl.ds(wid * GROUPS, GROUPS), :], w_v)
    plsc.subcore_barrier()

    def body(g, carry):
        pltpu.sync_copy(w_v.at[g], acc.at[idx_v.at[g]], add=True)
        return carry

    lax.fori_loop(0, GROUPS, body, 0)
    plsc.subcore_barrier()
    pltpu.sync_copy(acc.at[pl.ds(tb, DPT)],
                    out_hbm.at[pl.ds(c * DEG_ROWS + tb, DPT)])


_deg_call = pl.kernel(
    _deg_body,
    out_type=jax.ShapeDtypeStruct((NC * DEG_ROWS,), jnp.float32),
    mesh=_MESH,
    compiler_params=_SC_PARAMS,
    scratch_types=[
        pltpu.VMEM((GROUPS, G), jnp.int32),
        pltpu.VMEM((GROUPS, G), jnp.float32),
        pltpu.VMEM_SHARED((DEG_ROWS,), jnp.float32),
    ],
)


def _norm_body(src_hbm, dst_hbm, w_hbm, dinv_hbm, out_hbm,
               src_v, dst_v, w_v, dinv_v, norm_v):
    c = lax.axis_index("c")
    s = lax.axis_index("s")
    wid = c * NS + s
    gb = wid * GROUPS
    pltpu.sync_copy(dinv_hbm, dinv_v)
    pltpu.sync_copy(src_hbm.at[pl.ds(gb, GROUPS), :], src_v)
    pltpu.sync_copy(dst_hbm.at[pl.ds(gb, GROUPS), :], dst_v)
    pltpu.sync_copy(w_hbm.at[pl.ds(gb, GROUPS), :], w_v)

    def group(g, carry):
        for k in range(G // LANES):
            sl = pl.ds(k * LANES, LANES)
            nm = (plsc.load_gather(dinv_v, [src_v[g, sl]])
                  * plsc.load_gather(dinv_v, [dst_v[g, sl]])
                  * w_v[g, sl])
            norm_v[g, sl] = nm
        return carry

    lax.fori_loop(0, GROUPS, group, 0)
    pltpu.sync_copy(norm_v, out_hbm.at[pl.ds(gb, GROUPS), :])


_norm_call = pl.kernel(
    _norm_body,
    out_type=jax.ShapeDtypeStruct((TOT_G, G), jnp.float32),
    mesh=_MESH,
    compiler_params=_SC_PARAMS,
    scratch_types=[
        pltpu.VMEM((GROUPS, G), jnp.int32),
        pltpu.VMEM((GROUPS, G), jnp.int32),
        pltpu.VMEM((GROUPS, G), jnp.float32),
        pltpu.VMEM((DINV_PAD,), jnp.float32),
        pltpu.VMEM((GROUPS, G), jnp.float32),
    ],
)


def _agg_body(h_hbm, src_hbm, dst_hbm, norm_hbm, out_hbm,
              src_v, dst_v, norm_v,
              r0, r1, r2, r3, r4, acc,
              sg0, sg1, sg2, sg3, sg4,
              ss0, ss1, ss2, ss3, ss4,
              si0, si1, si2):
    c = lax.axis_index("c")
    s = lax.axis_index("s")
    tb = s * RPT

    zvec = jnp.zeros((LANES,), jnp.float32)

    def zrow(e, carry):
        for j in range(D // LANES):
            r0[e, pl.ds(j * LANES, LANES)] = zvec
        return carry

    lax.fori_loop(0, G2, zrow, 0)
    for k in range(RPT // G2):
        pltpu.sync_copy(r0, acc.at[pl.ds(tb + k * G2, G2), :])
    _rem = RPT - (RPT // G2) * G2
    if _rem:
        pltpu.sync_copy(r0.at[pl.ds(0, _rem), :],
                        acc.at[pl.ds(tb + (RPT // G2) * G2, _rem), :])
    plsc.subcore_barrier()

    rows = (r0, r1, r2, r3, r4)
    semg = (sg0, sg1, sg2, sg3, sg4)
    sems = (ss0, ss1, ss2, ss3, ss4)

    nch = jnp.where(c == 0, C0 // AIDX, C1 // AIDX)
    gstart = jnp.where(c == 0, s * C0, NS * C0 + s * C1)

    def chunk(cc, carry):
        gb = gstart + cc * AIDX
        d1 = pltpu.async_copy(src_hbm.at[pl.ds(gb, AIDX), :], src_v, si0)
        d2 = pltpu.async_copy(dst_hbm.at[pl.ds(gb, AIDX), :], dst_v, si1)
        d3 = pltpu.async_copy(norm_hbm.at[pl.ds(gb, AIDX), :], norm_v, si2)
        d1.wait()
        d2.wait()
        d3.wait()

        gdesc = [None] * NBUF
        sdesc = [None] * NBUF
        for g in range(PRE):
            gdesc[g] = pltpu.async_copy(h_hbm.at[src_v.at[g]], rows[g],
                                        semg[g])
        for g in range(AIDX):
            b = g % NBUF
            gn = g + PRE
            if gn < AIDX:
                bb = gn % NBUF
                if sdesc[bb] is not None:
                    sdesc[bb].wait()
                gdesc[bb] = pltpu.async_copy(h_hbm.at[src_v.at[gn]],
                                             rows[bb], semg[bb])
            gdesc[b].wait()
            rv = rows[b]
            gsplat = jnp.full((LANES,), g, jnp.int32)

            def scale(e, cc2, rv=rv, gsplat=gsplat):
                nsp = plsc.load_gather(
                    norm_v, [gsplat, jnp.full((LANES,), e, jnp.int32)])
                for j in range(D // LANES):
                    sj = pl.ds(j * LANES, LANES)
                    rv[e, sj] = rv[e, sj] * nsp
                return cc2

            lax.fori_loop(0, G2, scale, 0)
            sdesc[b] = pltpu.async_copy(rv, acc.at[dst_v.at[g]], sems[b],
                                        add=True)
        for b in range(NBUF):
            if sdesc[b] is not None:
                sdesc[b].wait()
        return carry

    lax.fori_loop(0, nch, chunk, 0)
    plsc.subcore_barrier()
    pltpu.sync_copy(acc.at[pl.ds(tb, RPT), :], out_hbm.at[c, pl.ds(tb, RPT), :])


_agg_call = pl.kernel(
    _agg_body,
    out_type=jax.ShapeDtypeStruct((NC, ACC_ROWS, D), jnp.float32),
    mesh=_MESH,
    compiler_params=_SC_PARAMS,
    scratch_types=(
        [pltpu.VMEM((AIDX, G2), jnp.int32),
         pltpu.VMEM((AIDX, G2), jnp.int32),
         pltpu.VMEM((AIDX, G2), jnp.float32)]
        + [pltpu.VMEM((G2, D), jnp.float32)] * NBUF
        + [pltpu.VMEM_SHARED((ACC_ROWS, D), jnp.float32)]
        + [pltpu.SemaphoreType.DMA] * (2 * NBUF + 3)
    ),
)


def _pre_body(x_ref, w1_ref, deg_ref, h_ref, dinv_ref):
    h_ref[...] = jnp.dot(x_ref[...], w1_ref[...],
                         preferred_element_type=jnp.float32)
    dtot = deg_ref[0] + deg_ref[1] + 2.0
    dinv_ref[...] = jnp.where(dtot > 0,
                              lax.rsqrt(jnp.maximum(dtot, 1e-12)),
                              0.0)


_pre = pl.pallas_call(
    _pre_body,
    grid=(N // RB,),
    in_specs=[
        pl.BlockSpec((RB, D), lambda i: (i, 0)),
        pl.BlockSpec((D, D), lambda i: (0, 0)),
        pl.BlockSpec((NC, RB, 1), lambda i: (0, i, 0)),
    ],
    out_specs=[
        pl.BlockSpec((RB, D), lambda i: (i, 0)),
        pl.BlockSpec((RB, 1), lambda i: (i, 0)),
    ],
    out_shape=[
        jax.ShapeDtypeStruct((N, D), jnp.float32),
        jax.ShapeDtypeStruct((N, 1), jnp.float32),
    ],
)


def _mid_body(acc_ref, h_ref, dinv_ref, b_ref, w2_ref, out_ref):
    dinv = dinv_ref[...]
    z = (acc_ref[0] + acc_ref[1]
         + 2.0 * dinv * dinv * h_ref[...]
         + b_ref[...])
    out_ref[...] = jnp.dot(jnp.maximum(z, 0.0), w2_ref[...],
                           preferred_element_type=jnp.float32)


_mid = pl.pallas_call(
    _mid_body,
    grid=(N // RB,),
    in_specs=[
        pl.BlockSpec((NC, RB, D), lambda i: (0, i, 0)),
        pl.BlockSpec((RB, D), lambda i: (i, 0)),
        pl.BlockSpec((RB, 1), lambda i: (i, 0)),
        pl.BlockSpec((1, D), lambda i: (0, 0)),
        pl.BlockSpec((D, D), lambda i: (0, 0)),
    ],
    out_specs=pl.BlockSpec((RB, D), lambda i: (i, 0)),
    out_shape=jax.ShapeDtypeStruct((N, D), jnp.float32),
)


def _fin_body(acc_ref, h_ref, dinv_ref, b_ref, wfc_ref, bfc_ref, scale_ref,
              out_ref):
    dinv = dinv_ref[...]
    z = (acc_ref[0] + acc_ref[1]
         + 2.0 * dinv * dinv * h_ref[...]
         + b_ref[...])
    hp = jnp.maximum(z, 0.0) * scale_ref[0, 0]
    out_ref[...] = jnp.dot(hp, wfc_ref[...],
                           preferred_element_type=jnp.float32) + bfc_ref[...]


_fin = pl.pallas_call(
    _fin_body,
    grid=(N // RB,),
    in_specs=[
        pl.BlockSpec((NC, RB, D), lambda i: (0, i, 0)),
        pl.BlockSpec((RB, D), lambda i: (i, 0)),
        pl.BlockSpec((RB, 1), lambda i: (i, 0)),
        pl.BlockSpec((1, D), lambda i: (0, 0)),
        pl.BlockSpec((D, 1), lambda i: (0, 0)),
        pl.BlockSpec((1, 1), lambda i: (0, 0)),
        pl.BlockSpec((1, 1), lambda i: (0, 0)),
    ],
    out_specs=pl.BlockSpec((RB, 1), lambda i: (i, 0)),
    out_shape=jax.ShapeDtypeStruct((N, 1), jnp.float32),
)


def kernel(x, edge_index, edge_weight, prob, W1, b1, W2, b2, Wfc, bfc):
    src = edge_index[0]
    dst = edge_index[1]
    pad = E_PAD - E
    srcp = jnp.concatenate(
        [src, jnp.zeros((pad,), jnp.int32)]).reshape(TOT_G, G)
    dstp = jnp.concatenate(
        [dst, jnp.full((pad,), N, jnp.int32)]).reshape(TOT_G, G)
    wp = jnp.concatenate(
        [edge_weight, jnp.zeros((pad,), jnp.float32)]).reshape(TOT_G, G)
    zrow = jnp.zeros((DEG_ROWS,), jnp.float32)

    deg_flat = _deg_call(dstp, wp, zrow)
    deg3 = deg_flat.reshape(NC, DEG_ROWS)[:, :N].reshape(NC, N, 1)

    h1, dinv = _pre(x, W1, deg3)
    dinv_flat = jnp.concatenate(
        [dinv.reshape(N), jnp.zeros((DINV_PAD - N,), jnp.float32)])

    normp = _norm_call(srcp, dstp, wp, dinv_flat)
    src2 = srcp.reshape(TOT_G2, G2)
    dst2 = dstp.reshape(TOT_G2, G2)
    norm2 = normp.reshape(TOT_G2, G2)

    acc1 = _agg_call(h1, src2, dst2, norm2)[:, :N]
    h2 = _mid(acc1, h1, dinv, b1.reshape(1, D), W2)
    acc2 = _agg_call(h2, src2, dst2, norm2)[:, :N]

    scale = (1.0 / (1.0 - jnp.asarray(prob, jnp.float32))).reshape(1, 1)
    out = _fin(acc2, h2, dinv, b2.reshape(1, D), Wfc,
               bfc.reshape(1, 1), scale)
    return out


# speedup vs baseline: 1.0382x; 1.0382x over previous
import jax
import jax.numpy as jnp
from jax import lax
from jax.experimental import pallas as pl
from jax.experimental.pallas import tpu as pltpu
from jax.experimental.pallas import tpu_sc as plsc

N = 10000
E = 320000
D = 128

NC = 2
NS = 16
NW = NC * NS
LANES = 16

G = 128
GROUPS = 80
IDX_CH = 8
EPW = G * GROUPS
E_PAD = EPW * NW
TOT_G = E_PAD // G
G2 = 64
TOT_G2 = E_PAD // G2
AIDX = 16
NBUF = 5
PRE = NBUF - 1
C0 = 240
C1 = 80

ACC_ROWS = 10112
RPT = ACC_ROWS // NS
DEG_ROWS = 16384
DPT = DEG_ROWS // NS
DINV_PAD = N + 16

RB = 1000


_MESH = plsc.VectorSubcoreMesh(core_axis_name="c", subcore_axis_name="s")
_SC_PARAMS = pltpu.CompilerParams(needs_layout_passes=False)


def _deg_body(dst_hbm, w_hbm, zrow_hbm, out_hbm, idx_v, w_v, acc):
    c = lax.axis_index("c")
    s = lax.axis_index("s")
    wid = c * NS + s
    tb = s * DPT
    pltpu.sync_copy(zrow_hbm.at[pl.ds(tb, DPT)], acc.at[pl.ds(tb, DPT)])
    pltpu.sync_copy(dst_hbm.at[pl.ds(wid * GROUPS, GROUPS), :], idx_v)
    pltpu.sync_copy(w_hbm.at[pl.ds(wid * GROUPS, GROUPS), :], w_v)
    plsc.subcore_barrier()

    def body(g, carry):
        pltpu.sync_copy(w_v.at[g], acc.at[idx_v.at[g]], add=True)
        return carry

    lax.fori_loop(0, GROUPS, body, 0)
    plsc.subcore_barrier()
    pltpu.sync_copy(acc.at[pl.ds(tb, DPT)],
                    out_hbm.at[pl.ds(c * DEG_ROWS + tb, DPT)])


_deg_call = pl.kernel(
    _deg_body,
    out_type=jax.ShapeDtypeStruct((NC * DEG_ROWS,), jnp.float32),
    mesh=_MESH,
    compiler_params=_SC_PARAMS,
    scratch_types=[
        pltpu.VMEM((GROUPS, G), jnp.int32),
        pltpu.VMEM((GROUPS, G), jnp.float32),
        pltpu.VMEM_SHARED((DEG_ROWS,), jnp.float32),
    ],
)


def _norm_body(src_hbm, dst_hbm, w_hbm, deg_hbm, out_hbm, dinv_hbm,
               src_v, dst_v, w_v, d0_v, d1_v, dinv_v, norm_v):
    c = lax.axis_index("c")
    s = lax.axis_index("s")
    wid = c * NS + s
    gb = wid * GROUPS
    pltpu.sync_copy(deg_hbm.at[pl.ds(0, DINV_PAD)], d0_v)
    pltpu.sync_copy(deg_hbm.at[pl.ds(DEG_ROWS, DINV_PAD)], d1_v)
    pltpu.sync_copy(src_hbm.at[pl.ds(gb, GROUPS), :], src_v)
    pltpu.sync_copy(dst_hbm.at[pl.ds(gb, GROUPS), :], dst_v)
    pltpu.sync_copy(w_hbm.at[pl.ds(gb, GROUPS), :], w_v)

    def rsq(i, carry):
        sl = pl.ds(i * LANES, LANES)
        dt = d0_v[sl] + d1_v[sl] + 2.0
        yi = jnp.int32(0x5F3759DF) - (plsc.bitcast(dt, jnp.int32) >> 1)
        y = plsc.bitcast(yi, jnp.float32)
        for _ in range(3):
            y = y * (1.5 - 0.5 * dt * y * y)
        dinv_v[sl] = y
        return carry

    lax.fori_loop(0, DINV_PAD // LANES, rsq, 0)

    def group(g, carry):
        for k in range(G // LANES):
            sl = pl.ds(k * LANES, LANES)
            nm = (plsc.load_gather(dinv_v, [src_v[g, sl]])
                  * plsc.load_gather(dinv_v, [dst_v[g, sl]])
                  * w_v[g, sl])
            norm_v[g, sl] = nm
        return carry

    lax.fori_loop(0, GROUPS, group, 0)
    pltpu.sync_copy(norm_v, out_hbm.at[pl.ds(gb, GROUPS), :])

    @pl.when(jnp.logical_and(c == 0, s == 0))
    def _():
        pltpu.sync_copy(dinv_v, dinv_hbm)


_norm_call = pl.kernel(
    _norm_body,
    out_type=(jax.ShapeDtypeStruct((TOT_G, G), jnp.float32),
              jax.ShapeDtypeStruct((DINV_PAD,), jnp.float32)),
    mesh=_MESH,
    compiler_params=_SC_PARAMS,
    scratch_types=[
        pltpu.VMEM((GROUPS, G), jnp.int32),
        pltpu.VMEM((GROUPS, G), jnp.int32),
        pltpu.VMEM((GROUPS, G), jnp.float32),
        pltpu.VMEM((DINV_PAD,), jnp.float32),
        pltpu.VMEM((DINV_PAD,), jnp.float32),
        pltpu.VMEM((DINV_PAD,), jnp.float32),
        pltpu.VMEM((GROUPS, G), jnp.float32),
    ],
)


def _agg_body(h_hbm, src_hbm, dst_hbm, norm_hbm, out_hbm,
              src_v, dst_v, norm_v,
              r0, r1, r2, r3, r4, acc,
              sg0, sg1, sg2, sg3, sg4,
              ss0, ss1, ss2, ss3, ss4,
              si0, si1, si2):
    c = lax.axis_index("c")
    s = lax.axis_index("s")
    tb = s * RPT

    zvec = jnp.zeros((LANES,), jnp.float32)

    def zrow(e, carry):
        for j in range(D // LANES):
            r0[e, pl.ds(j * LANES, LANES)] = zvec
        return carry

    lax.fori_loop(0, G2, zrow, 0)
    for k in range(RPT // G2):
        pltpu.sync_copy(r0, acc.at[pl.ds(tb + k * G2, G2), :])
    _rem = RPT - (RPT // G2) * G2
    if _rem:
        pltpu.sync_copy(r0.at[pl.ds(0, _rem), :],
                        acc.at[pl.ds(tb + (RPT // G2) * G2, _rem), :])
    plsc.subcore_barrier()

    rows = (r0, r1, r2, r3, r4)
    semg = (sg0, sg1, sg2, sg3, sg4)
    sems = (ss0, ss1, ss2, ss3, ss4)

    nch = jnp.where(c == 0, C0 // AIDX, C1 // AIDX)
    gstart = jnp.where(c == 0, s * C0, NS * C0 + s * C1)

    def chunk(cc, carry):
        gb = gstart + cc * AIDX
        d1 = pltpu.async_copy(src_hbm.at[pl.ds(gb, AIDX), :], src_v, si0)
        d2 = pltpu.async_copy(dst_hbm.at[pl.ds(gb, AIDX), :], dst_v, si1)
        d3 = pltpu.async_copy(norm_hbm.at[pl.ds(gb, AIDX), :], norm_v, si2)
        d1.wait()
        d2.wait()
        d3.wait()

        gdesc = [None] * NBUF
        sdesc = [None] * NBUF
        for g in range(PRE):
            gdesc[g] = pltpu.async_copy(h_hbm.at[src_v.at[g]], rows[g],
                                        semg[g])
        for g in range(AIDX):
            b = g % NBUF
            gn = g + PRE
            if gn < AIDX:
                bb = gn % NBUF
                if sdesc[bb] is not None:
                    sdesc[bb].wait()
                gdesc[bb] = pltpu.async_copy(h_hbm.at[src_v.at[gn]],
                                             rows[bb], semg[bb])
            gdesc[b].wait()
            rv = rows[b]
            gsplat = jnp.full((LANES,), g, jnp.int32)

            def scale(e, cc2, rv=rv, gsplat=gsplat):
                nsp = plsc.load_gather(
                    norm_v, [gsplat, jnp.full((LANES,), e, jnp.int32)])
                for j in range(D // LANES):
                    sj = pl.ds(j * LANES, LANES)
                    rv[e, sj] = rv[e, sj] * nsp
                return cc2

            lax.fori_loop(0, G2, scale, 0)
            sdesc[b] = pltpu.async_copy(rv, acc.at[dst_v.at[g]], sems[b],
                                        add=True)
        for b in range(NBUF):
            if sdesc[b] is not None:
                sdesc[b].wait()
        return carry

    lax.fori_loop(0, nch, chunk, 0)
    plsc.subcore_barrier()
    pltpu.sync_copy(acc.at[pl.ds(tb, RPT), :], out_hbm.at[c, pl.ds(tb, RPT), :])


_agg_call = pl.kernel(
    _agg_body,
    out_type=jax.ShapeDtypeStruct((NC, ACC_ROWS, D), jnp.float32),
    mesh=_MESH,
    compiler_params=_SC_PARAMS,
    scratch_types=(
        [pltpu.VMEM((AIDX, G2), jnp.int32),
         pltpu.VMEM((AIDX, G2), jnp.int32),
         pltpu.VMEM((AIDX, G2), jnp.float32)]
        + [pltpu.VMEM((G2, D), jnp.float32)] * NBUF
        + [pltpu.VMEM_SHARED((ACC_ROWS, D), jnp.float32)]
        + [pltpu.SemaphoreType.DMA] * (2 * NBUF + 3)
    ),
)


def _pre_body(x_ref, w1_ref, h_ref):
    h_ref[...] = jnp.dot(x_ref[...], w1_ref[...],
                         preferred_element_type=jnp.float32)


_pre = pl.pallas_call(
    _pre_body,
    grid=(N // RB,),
    in_specs=[
        pl.BlockSpec((RB, D), lambda i: (i, 0)),
        pl.BlockSpec((D, D), lambda i: (0, 0)),
    ],
    out_specs=pl.BlockSpec((RB, D), lambda i: (i, 0)),
    out_shape=jax.ShapeDtypeStruct((N, D), jnp.float32),
)


def _mid_body(acc_ref, h_ref, dinv_ref, b_ref, w2_ref, out_ref):
    dinv = dinv_ref[...]
    z = (acc_ref[0] + acc_ref[1]
         + 2.0 * dinv * dinv * h_ref[...]
         + b_ref[...])
    out_ref[...] = jnp.dot(jnp.maximum(z, 0.0), w2_ref[...],
                           preferred_element_type=jnp.float32)


_mid = pl.pallas_call(
    _mid_body,
    grid=(N // RB,),
    in_specs=[
        pl.BlockSpec((NC, RB, D), lambda i: (0, i, 0)),
        pl.BlockSpec((RB, D), lambda i: (i, 0)),
        pl.BlockSpec((RB, 1), lambda i: (i, 0)),
        pl.BlockSpec((1, D), lambda i: (0, 0)),
        pl.BlockSpec((D, D), lambda i: (0, 0)),
    ],
    out_specs=pl.BlockSpec((RB, D), lambda i: (i, 0)),
    out_shape=jax.ShapeDtypeStruct((N, D), jnp.float32),
)


def _fin_body(acc_ref, h_ref, dinv_ref, b_ref, wfc_ref, bfc_ref, scale_ref,
              out_ref):
    dinv = dinv_ref[...]
    z = (acc_ref[0] + acc_ref[1]
         + 2.0 * dinv * dinv * h_ref[...]
         + b_ref[...])
    hp = jnp.maximum(z, 0.0) * scale_ref[0, 0]
    out_ref[...] = jnp.dot(hp, wfc_ref[...],
                           preferred_element_type=jnp.float32) + bfc_ref[...]


_fin = pl.pallas_call(
    _fin_body,
    grid=(N // RB,),
    in_specs=[
        pl.BlockSpec((NC, RB, D), lambda i: (0, i, 0)),
        pl.BlockSpec((RB, D), lambda i: (i, 0)),
        pl.BlockSpec((RB, 1), lambda i: (i, 0)),
        pl.BlockSpec((1, D), lambda i: (0, 0)),
        pl.BlockSpec((D, 1), lambda i: (0, 0)),
        pl.BlockSpec((1, 1), lambda i: (0, 0)),
        pl.BlockSpec((1, 1), lambda i: (0, 0)),
    ],
    out_specs=pl.BlockSpec((RB, 1), lambda i: (i, 0)),
    out_shape=jax.ShapeDtypeStruct((N, 1), jnp.float32),
)


def kernel(x, edge_index, edge_weight, prob, W1, b1, W2, b2, Wfc, bfc):
    src = edge_index[0]
    dst = edge_index[1]
    pad = E_PAD - E
    srcp = jnp.concatenate(
        [src, jnp.zeros((pad,), jnp.int32)]).reshape(TOT_G, G)
    dstp = jnp.concatenate(
        [dst, jnp.full((pad,), N, jnp.int32)]).reshape(TOT_G, G)
    wp = jnp.concatenate(
        [edge_weight, jnp.zeros((pad,), jnp.float32)]).reshape(TOT_G, G)
    zrow = jnp.zeros((DEG_ROWS,), jnp.float32)

    deg_flat = _deg_call(dstp, wp, zrow)
    h1 = _pre(x, W1)

    normp, dinv_pad = _norm_call(srcp, dstp, wp, deg_flat)
    dinv = dinv_pad[:N].reshape(N, 1)
    src2 = srcp.reshape(TOT_G2, G2)
    dst2 = dstp.reshape(TOT_G2, G2)
    norm2 = normp.reshape(TOT_G2, G2)

    acc1 = _agg_call(h1, src2, dst2, norm2)[:, :N]
    h2 = _mid(acc1, h1, dinv, b1.reshape(1, D), W2)
    acc2 = _agg_call(h2, src2, dst2, norm2)[:, :N]

    scale = (1.0 / (1.0 - jnp.asarray(prob, jnp.float32))).reshape(1, 1)
    out = _fin(acc2, h2, dinv, b2.reshape(1, D), Wfc,
               bfc.reshape(1, 1), scale)
    return out
